# baseline (speedup 1.0000x reference)
import functools

import jax
import jax.numpy as jnp
from jax import lax
from jax.experimental import pallas as pl
from jax.experimental.pallas import tpu as pltpu
from jax.experimental.pallas import tpu_sc as plsc

_ROWS, _N = 128, 32768
_NW = 32
_RPW = _ROWS // _NW
_NVROW = _N // 16
_BLK = 64
_CAP = 8192
_INT_MIN = -(2 ** 31)
_FLIP = 0x7FFFFFFF


def _key_s(v):
    b = plsc.bitcast(v, jnp.int32)
    return jnp.where(b >= 0, b, b ^ jnp.int32(_FLIP))


def _gather16(x, idx):
    dnums = lax.GatherDimensionNumbers(
        offset_dims=(), collapsed_slice_dims=(0,), start_index_map=(0,))
    return lax.gather(x, idx[:, None], dnums, (1,),
                      mode=lax.GatherScatterMode.PROMISE_IN_BOUNDS)


def _xsum(x):
    lane = lax.iota(jnp.int32, 16)
    for d in (1, 2, 4, 8):
        x = x + _gather16(x, lane ^ d)
    return x


def _xmax(x):
    lane = lax.iota(jnp.int32, 16)
    for d in (1, 2, 4, 8):
        x = jnp.maximum(x, _gather16(x, lane ^ d))
    return x


def _radix_kth_key(ibuf, nv, kk_v):
    int_min = jnp.int32(_INT_MIN)

    def bit_step(bi, prefix_u_v):
        bit_v = jnp.zeros((16,), jnp.int32) + (jnp.int32(1) << (31 - bi))
        cand_u_v = prefix_u_v | bit_v
        cand_s_v = cand_u_v ^ int_min

        def cnt_step(j, acc):
            v = ibuf[pl.ds(j * 16, 16)]
            return acc + jnp.where(v >= cand_s_v, 1, 0)

        acc = plsc.parallel_loop(0, nv, 1, unroll=4,
                                 carry=jnp.zeros((16,), jnp.int32))(cnt_step)
        cnt_v = _xsum(acc)
        return jnp.where(cnt_v >= kk_v, cand_u_v, prefix_u_v)

    prefix_u_v = lax.fori_loop(0, 32, bit_step, jnp.zeros((16,), jnp.int32))
    return prefix_u_v ^ int_min


def _sc_thresholds(scores, k_arr):
    mesh = plsc.VectorSubcoreMesh(core_axis_name="c", subcore_axis_name="s",
                                  num_cores=2, num_subcores=16)

    @functools.partial(
        pl.kernel,
        out_type=jax.ShapeDtypeStruct((_NW, 16), jnp.float32),
        mesh=mesh,
        compiler_params=pltpu.CompilerParams(needs_layout_passes=False),
        scratch_types=[
            pltpu.VMEM((2 * _N,), jnp.float32),
            pltpu.VMEM((_CAP,), jnp.float32),
            pltpu.VMEM((_CAP,), jnp.int32),
            pltpu.VMEM((16,), jnp.int32),
            pltpu.VMEM((16,), jnp.int32),
            pltpu.VMEM((16,), jnp.float32),
            pltpu.VMEM((16,), jnp.float32),
            pltpu.SemaphoreType.DMA,
            pltpu.SemaphoreType.DMA,
        ],
    )
    def sc_kernel(scores_hbm, k_hbm, out_hbm, rowbufs, cbuf, ibuf, kbuf,
                  cntref, tref, tbuf, sem0, sem1):
        neg_inf = jnp.float32(-jnp.inf)
        int_min = jnp.int32(_INT_MIN)
        lane = lax.iota(jnp.int32, 16)
        wid = lax.axis_index("s") * 2 + lax.axis_index("c")

        pltpu.sync_copy(k_hbm, kbuf)
        kk_v = kbuf[...]


        def select_kth(c16, t):
            nv = lax.shift_right_logical(_xmax(c16)[0], 4)

            def keyfill(j, _):
                v = cbuf[pl.ds(j * 16, 16)]
                valid = (j * 16) < c16
                ibuf[pl.ds(j * 16, 16)] = jnp.where(valid, _key_s(v),
                                                    int_min)
                return 0

            plsc.parallel_loop(0, nv, 1, unroll=4,
                               carry=jnp.int32(0))(keyfill)
            ts_v = _radix_kth_key(ibuf, nv, kk_v)
            tf_v = plsc.bitcast(
                jnp.where(ts_v >= 0, ts_v, ts_v ^ jnp.int32(_FLIP)),
                jnp.float32)
            tf_v = jnp.where(ts_v == int_min, neg_inf, tf_v)
            return jnp.maximum(t, tf_v)

        sems = (sem0, sem1)
        handles = [None, None]
        handles[0] = pltpu.async_copy(scores_hbm.at[wid * _RPW],
                                      rowbufs.at[pl.ds(0, _N)], sems[0])
        t_acc = jnp.zeros((16,), jnp.float32)
        for rr in range(_RPW):
            b = rr % 2
            handles[b].wait()
            if rr + 1 < _RPW:
                handles[1 - b] = pltpu.async_copy(
                    scores_hbm.at[wid * _RPW + rr + 1],
                    rowbufs.at[pl.ds((1 - b) * _N, _N)], sems[1 - b])
            rowbuf = rowbufs.at[pl.ds(b * _N, _N)]

            def filt_block(blk, carry):
                c16, t = carry

                def append(i, c16):
                    v = rowbuf[pl.ds(i * 16, 16)]
                    m = v > t
                    plsc.store_scatter(cbuf, [c16 + lane], v, mask=m)
                    return c16 + jnp.where(m, 16, 0)

                c16 = plsc.parallel_loop(blk * _BLK, (blk + 1) * _BLK, 1,
                                         unroll=8, carry=c16)(append)

                cntref[...] = c16
                tref[...] = t

                @pl.when((blk == 0) | (_xmax(c16)[0] > _CAP - _BLK * 16))
                def _rebuild():
                    t_new = select_kth(c16, t)
                    nv = lax.shift_right_logical(_xmax(c16)[0], 4)

                    def compact(j, c16n):
                        v = cbuf[pl.ds(j * 16, 16)]
                        m = ((j * 16) < c16) & (v > t_new)
                        plsc.store_scatter(cbuf, [c16n + lane], v, mask=m)
                        return c16n + jnp.where(m, 16, 0)

                    cntref[...] = lax.fori_loop(0, nv, compact,
                                                jnp.zeros((16,), jnp.int32))
                    tref[...] = t_new

                return cntref[...], tref[...]

            init = (jnp.zeros((16,), jnp.int32),
                    jnp.full((16,), neg_inf, jnp.float32))
            c16, t = lax.fori_loop(0, _NVROW // _BLK, filt_block, init)
            t_fin = select_kth(c16, t)
            t_acc = jnp.where(lane == rr, t_fin, t_acc)

        tbuf[...] = t_acc
        pltpu.sync_copy(tbuf, out_hbm.at[wid])

    return sc_kernel(scores, k_arr)


def _sm_body(x_ref, t_ref, o_ref):
    x = x_ref[...]
    t = t_ref[...]
    mask = x >= t
    m = jnp.max(x, axis=1, keepdims=True)
    e = jnp.where(mask, jnp.exp(x - m), 0.0)
    o_ref[...] = e / jnp.sum(e, axis=1, keepdims=True)


def _tc_softmax(scores, thresh):
    rows, n = scores.shape
    r_blk = 16
    return pl.pallas_call(
        _sm_body,
        grid=(rows // r_blk,),
        in_specs=[
            pl.BlockSpec((r_blk, n), lambda i: (i, 0)),
            pl.BlockSpec((r_blk, 1), lambda i: (i, 0)),
        ],
        out_specs=pl.BlockSpec((r_blk, n), lambda i: (i, 0)),
        out_shape=jax.ShapeDtypeStruct(scores.shape, scores.dtype),
    )(scores, thresh)


def kernel(scores, k):
    k_arr = jnp.full((16,), k, jnp.int32)
    th = _sc_thresholds(scores, k_arr)
    thresh = th[:, :_RPW].reshape(_ROWS, 1)
    return _tc_softmax(scores, thresh)


# speedup vs baseline: 1.3972x; 1.3972x over previous
import functools

import jax
import jax.numpy as jnp
from jax import lax
from jax.experimental import pallas as pl
from jax.experimental.pallas import tpu as pltpu
from jax.experimental.pallas import tpu_sc as plsc

_ROWS, _N = 128, 32768
_NW = 32
_RPW = _ROWS // _NW
_NVROW = _N // 16
_BLK = 128
_WU = 512
_CAP = 8192
_INT_MIN = -(2 ** 31)
_FLIP = 0x7FFFFFFF


def _key_s(v):
    b = plsc.bitcast(v, jnp.int32)
    return jnp.where(b >= 0, b, b ^ jnp.int32(_FLIP))


def _gather16(x, idx):
    dnums = lax.GatherDimensionNumbers(
        offset_dims=(), collapsed_slice_dims=(0,), start_index_map=(0,))
    return lax.gather(x, idx[:, None], dnums, (1,),
                      mode=lax.GatherScatterMode.PROMISE_IN_BOUNDS)


def _xsum(x):
    lane = lax.iota(jnp.int32, 16)
    for d in (1, 2, 4, 8):
        x = x + _gather16(x, lane ^ d)
    return x


def _xmax(x):
    lane = lax.iota(jnp.int32, 16)
    for d in (1, 2, 4, 8):
        x = jnp.maximum(x, _gather16(x, lane ^ d))
    return x


def _xmin(x):
    lane = lax.iota(jnp.int32, 16)
    for d in (1, 2, 4, 8):
        x = jnp.minimum(x, _gather16(x, lane ^ d))
    return x


def _radix_kth_key(ibuf, nv, kk_v):
    int_min = jnp.int32(_INT_MIN)

    def bit_step(bi, prefix_u_v):
        bit_v = jnp.zeros((16,), jnp.int32) + (jnp.int32(1) << (31 - bi))
        cand_u_v = prefix_u_v | bit_v
        cand_s_v = cand_u_v ^ int_min

        def cnt_step(j, acc):
            v = ibuf[pl.ds(j * 16, 16)]
            return acc + jnp.where(v >= cand_s_v, 1, 0)

        acc = plsc.parallel_loop(0, nv, 1, unroll=4,
                                 carry=jnp.zeros((16,), jnp.int32))(cnt_step)
        cnt_v = _xsum(acc)
        return jnp.where(cnt_v >= kk_v, cand_u_v, prefix_u_v)

    prefix_u_v = lax.fori_loop(0, 32, bit_step, jnp.zeros((16,), jnp.int32))
    return prefix_u_v ^ int_min


def _sc_thresholds(scores, k_arr):
    mesh = plsc.VectorSubcoreMesh(core_axis_name="c", subcore_axis_name="s",
                                  num_cores=2, num_subcores=16)

    @functools.partial(
        pl.kernel,
        out_type=jax.ShapeDtypeStruct((_NW, 16), jnp.float32),
        mesh=mesh,
        compiler_params=pltpu.CompilerParams(needs_layout_passes=False),
        scratch_types=[
            pltpu.VMEM((2 * _N,), jnp.float32),
            pltpu.VMEM((_CAP,), jnp.float32),
            pltpu.VMEM((_CAP,), jnp.int32),
            pltpu.VMEM((16,), jnp.int32),
            pltpu.VMEM((16,), jnp.int32),
            pltpu.VMEM((16,), jnp.float32),
            pltpu.VMEM((16,), jnp.float32),
            pltpu.SemaphoreType.DMA,
            pltpu.SemaphoreType.DMA,
        ],
    )
    def sc_kernel(scores_hbm, k_hbm, out_hbm, rowbufs, cbuf, ibuf, kbuf,
                  cntref, tref, tbuf, sem0, sem1):
        neg_inf = jnp.float32(-jnp.inf)
        int_min = jnp.int32(_INT_MIN)
        lane = lax.iota(jnp.int32, 16)
        wid = lax.axis_index("s") * 2 + lax.axis_index("c")

        pltpu.sync_copy(k_hbm, kbuf)
        kk_v = kbuf[...]


        def select_kth(c16, t):
            nv = lax.shift_right_logical(_xmax(c16)[0], 4)

            def keyfill(j, _):
                v = cbuf[pl.ds(j * 16, 16)]
                valid = (j * 16) < c16
                ibuf[pl.ds(j * 16, 16)] = jnp.where(valid, _key_s(v),
                                                    int_min)
                return 0

            plsc.parallel_loop(0, nv, 1, unroll=4,
                               carry=jnp.int32(0))(keyfill)
            ts_v = _radix_kth_key(ibuf, nv, kk_v)
            tf_v = plsc.bitcast(
                jnp.where(ts_v >= 0, ts_v, ts_v ^ jnp.int32(_FLIP)),
                jnp.float32)
            tf_v = jnp.where(ts_v == int_min, neg_inf, tf_v)
            return jnp.maximum(t, tf_v)

        sems = (sem0, sem1)
        handles = [None, None]
        handles[0] = pltpu.async_copy(scores_hbm.at[wid * _RPW],
                                      rowbufs.at[pl.ds(0, _N)], sems[0])
        t_acc = jnp.zeros((16,), jnp.float32)
        for rr in range(_RPW):
            b = rr % 2
            handles[b].wait()
            if rr + 1 < _RPW:
                handles[1 - b] = pltpu.async_copy(
                    scores_hbm.at[wid * _RPW + rr + 1],
                    rowbufs.at[pl.ds((1 - b) * _N, _N)], sems[1 - b])
            rowbuf = rowbufs.at[pl.ds(b * _N, _N)]

            def wu_step(i, ms):
                v = rowbuf[pl.ds(i * 16, 16)]
                m1, m2, m3, m4 = ms
                t1 = jnp.maximum(m1, v)
                b1 = jnp.minimum(m1, v)
                t2 = jnp.maximum(m2, b1)
                b2 = jnp.minimum(m2, b1)
                t3 = jnp.maximum(m3, b2)
                b3 = jnp.minimum(m3, b2)
                t4 = jnp.maximum(m4, b3)
                return (t1, t2, t3, t4)

            ms0 = (jnp.full((16,), neg_inf, jnp.float32),) * 4
            _, _, _, m4 = plsc.parallel_loop(0, _WU, 1, unroll=4,
                                             carry=ms0)(wu_step)
            t0 = _xmin(m4)

            def filt_block(blk, carry):
                c16, t = carry

                def append(i, c16):
                    v = rowbuf[pl.ds(i * 16, 16)]
                    m = v > t
                    plsc.store_scatter(cbuf, [c16 + lane], v, mask=m)
                    return c16 + jnp.where(m, 16, 0)

                c16 = plsc.parallel_loop(blk * _BLK, (blk + 1) * _BLK, 1,
                                         unroll=8, carry=c16)(append)

                cntref[...] = c16
                tref[...] = t

                @pl.when(_xmax(c16)[0] > _CAP - _BLK * 16)
                def _rebuild():
                    t_new = select_kth(c16, t)
                    nv = lax.shift_right_logical(_xmax(c16)[0], 4)

                    def compact(j, c16n):
                        v = cbuf[pl.ds(j * 16, 16)]
                        m = ((j * 16) < c16) & (v > t_new)
                        plsc.store_scatter(cbuf, [c16n + lane], v, mask=m)
                        return c16n + jnp.where(m, 16, 0)

                    cntref[...] = lax.fori_loop(0, nv, compact,
                                                jnp.zeros((16,), jnp.int32))
                    tref[...] = t_new

                return cntref[...], tref[...]

            init = (jnp.zeros((16,), jnp.int32), t0)
            c16, t = lax.fori_loop(0, _NVROW // _BLK, filt_block, init)
            t_fin = select_kth(c16, t)
            t_acc = jnp.where(lane == rr, t_fin, t_acc)

        tbuf[...] = t_acc
        pltpu.sync_copy(tbuf, out_hbm.at[wid])

    return sc_kernel(scores, k_arr)


def _sm_body(x_ref, t_ref, o_ref):
    x = x_ref[...]
    t = t_ref[...]
    mask = x >= t
    m = jnp.max(x, axis=1, keepdims=True)
    e = jnp.where(mask, jnp.exp(x - m), 0.0)
    o_ref[...] = e / jnp.sum(e, axis=1, keepdims=True)


def _tc_softmax(scores, thresh):
    rows, n = scores.shape
    r_blk = 16
    return pl.pallas_call(
        _sm_body,
        grid=(rows // r_blk,),
        in_specs=[
            pl.BlockSpec((r_blk, n), lambda i: (i, 0)),
            pl.BlockSpec((r_blk, 1), lambda i: (i, 0)),
        ],
        out_specs=pl.BlockSpec((r_blk, n), lambda i: (i, 0)),
        out_shape=jax.ShapeDtypeStruct(scores.shape, scores.dtype),
    )(scores, thresh)


def kernel(scores, k):
    k_arr = jnp.full((16,), k, jnp.int32)
    th = _sc_thresholds(scores, k_arr)
    thresh = th[:, :_RPW].reshape(_ROWS, 1)
    return _tc_softmax(scores, thresh)


# speedup vs baseline: 1.4588x; 1.0442x over previous
import functools

import jax
import jax.numpy as jnp
from jax import lax
from jax.experimental import pallas as pl
from jax.experimental.pallas import tpu as pltpu
from jax.experimental.pallas import tpu_sc as plsc

_ROWS, _N = 128, 32768
_NW = 32
_RPW = _ROWS // _NW
_NVROW = _N // 16
_BLK = 128
_WU = 512
_CAP = 8192
_INT_MIN = -(2 ** 31)
_FLIP = 0x7FFFFFFF


def _key_s(v):
    b = plsc.bitcast(v, jnp.int32)
    return jnp.where(b >= 0, b, b ^ jnp.int32(_FLIP))


def _gather16(x, idx):
    dnums = lax.GatherDimensionNumbers(
        offset_dims=(), collapsed_slice_dims=(0,), start_index_map=(0,))
    return lax.gather(x, idx[:, None], dnums, (1,),
                      mode=lax.GatherScatterMode.PROMISE_IN_BOUNDS)


def _xsum(x):
    lane = lax.iota(jnp.int32, 16)
    for d in (1, 2, 4, 8):
        x = x + _gather16(x, lane ^ d)
    return x


def _xmax(x):
    lane = lax.iota(jnp.int32, 16)
    for d in (1, 2, 4, 8):
        x = jnp.maximum(x, _gather16(x, lane ^ d))
    return x


def _xmin(x):
    lane = lax.iota(jnp.int32, 16)
    for d in (1, 2, 4, 8):
        x = jnp.minimum(x, _gather16(x, lane ^ d))
    return x


def _radix_kth_key(ibuf, nv, kk_v):
    int_min = jnp.int32(_INT_MIN)

    def bit_step(bi, prefix_u_v):
        bit_v = jnp.zeros((16,), jnp.int32) + (jnp.int32(1) << (31 - bi))
        cand_u_v = prefix_u_v | bit_v
        cand_s_v = cand_u_v ^ int_min

        def cnt_step(j, acc):
            v = ibuf[pl.ds(j * 16, 16)]
            return acc + jnp.where(v >= cand_s_v, 1, 0)

        acc = plsc.parallel_loop(0, nv, 1, unroll=4,
                                 carry=jnp.zeros((16,), jnp.int32))(cnt_step)
        cnt_v = _xsum(acc)
        return jnp.where(cnt_v >= kk_v, cand_u_v, prefix_u_v)

    prefix_u_v = lax.fori_loop(0, 32, bit_step, jnp.zeros((16,), jnp.int32))
    return prefix_u_v ^ int_min


def _sc_thresholds(scores, k_arr):
    mesh = plsc.VectorSubcoreMesh(core_axis_name="c", subcore_axis_name="s",
                                  num_cores=2, num_subcores=16)

    @functools.partial(
        pl.kernel,
        out_type=jax.ShapeDtypeStruct((_NW, 16), jnp.float32),
        mesh=mesh,
        compiler_params=pltpu.CompilerParams(needs_layout_passes=False),
        scratch_types=[
            pltpu.VMEM((2 * _N,), jnp.float32),
            pltpu.VMEM((_CAP,), jnp.float32),
            pltpu.VMEM((_CAP,), jnp.int32),
            pltpu.VMEM((16,), jnp.int32),
            pltpu.VMEM((16,), jnp.int32),
            pltpu.VMEM((16,), jnp.float32),
            pltpu.VMEM((16,), jnp.float32),
            pltpu.SemaphoreType.DMA,
            pltpu.SemaphoreType.DMA,
        ],
    )
    def sc_kernel(scores_hbm, k_hbm, out_hbm, rowbufs, cbuf, ibuf, kbuf,
                  cntref, tref, tbuf, sem0, sem1):
        neg_inf = jnp.float32(-jnp.inf)
        int_min = jnp.int32(_INT_MIN)
        lane = lax.iota(jnp.int32, 16)
        wid = lax.axis_index("s") * 2 + lax.axis_index("c")

        pltpu.sync_copy(k_hbm, kbuf)
        kk_v = kbuf[...]


        def select_kth(c16, t):
            nv = lax.shift_right_logical(_xmax(c16)[0], 4)

            def keyfill(j, _):
                v = cbuf[pl.ds(j * 16, 16)]
                valid = (j * 16) < c16
                ibuf[pl.ds(j * 16, 16)] = jnp.where(valid, _key_s(v),
                                                    int_min)
                return 0

            plsc.parallel_loop(0, nv, 1, unroll=4,
                               carry=jnp.int32(0))(keyfill)
            ts_v = _radix_kth_key(ibuf, nv, kk_v)
            tf_v = plsc.bitcast(
                jnp.where(ts_v >= 0, ts_v, ts_v ^ jnp.int32(_FLIP)),
                jnp.float32)
            tf_v = jnp.where(ts_v == int_min, neg_inf, tf_v)
            return jnp.maximum(t, tf_v)

        sems = (sem0, sem1)
        handles = [None, None]
        handles[0] = pltpu.async_copy(scores_hbm.at[wid * _RPW],
                                      rowbufs.at[pl.ds(0, _N)], sems[0])
        t_acc = jnp.zeros((16,), jnp.float32)
        for rr in range(_RPW):
            b = rr % 2
            handles[b].wait()
            if rr + 1 < _RPW:
                handles[1 - b] = pltpu.async_copy(
                    scores_hbm.at[wid * _RPW + rr + 1],
                    rowbufs.at[pl.ds((1 - b) * _N, _N)], sems[1 - b])
            rowbuf = rowbufs.at[pl.ds(b * _N, _N)]

            def wu_step(i, ms):
                v = rowbuf[pl.ds(i * 16, 16)]
                m1, m2, m3, m4 = ms
                t1 = jnp.maximum(m1, v)
                b1 = jnp.minimum(m1, v)
                t2 = jnp.maximum(m2, b1)
                b2 = jnp.minimum(m2, b1)
                t3 = jnp.maximum(m3, b2)
                b3 = jnp.minimum(m3, b2)
                t4 = jnp.maximum(m4, b3)
                return (t1, t2, t3, t4)

            ms0 = (jnp.full((16,), neg_inf, jnp.float32),) * 4
            _, _, _, m4 = plsc.parallel_loop(0, _WU, 1, unroll=4,
                                             carry=ms0)(wu_step)
            t0 = _xmin(m4)

            def filt_block(blk, carry):
                c16, t = carry

                def append(i, c16):
                    v = rowbuf[pl.ds(i * 16, 16)]
                    m = v > t
                    plsc.store_scatter(cbuf, [c16 + lane], v, mask=m)
                    return c16 + jnp.where(m, 16, 0)

                c16 = plsc.parallel_loop(blk * _BLK, (blk + 1) * _BLK, 1,
                                         unroll=8, carry=c16)(append)

                cntref[...] = c16
                tref[...] = t

                @pl.when(_xmax(c16)[0] > _CAP - _BLK * 16)
                def _rebuild():
                    t_new = select_kth(c16, t)
                    nv = lax.shift_right_logical(_xmax(c16)[0], 4)

                    def compact(j, c16n):
                        v = cbuf[pl.ds(j * 16, 16)]
                        m = ((j * 16) < c16) & (v > t_new)
                        plsc.store_scatter(cbuf, [c16n + lane], v, mask=m)
                        return c16n + jnp.where(m, 16, 0)

                    cntref[...] = lax.fori_loop(0, nv, compact,
                                                jnp.zeros((16,), jnp.int32))
                    tref[...] = t_new

                return cntref[...], tref[...]

            init = (jnp.zeros((16,), jnp.int32), t0)
            c16, t = lax.fori_loop(0, _NVROW // _BLK, filt_block, init)
            t_fin = select_kth(c16, t)
            t_acc = jnp.where(lane == rr, t_fin, t_acc)

        tbuf[...] = t_acc
        pltpu.sync_copy(tbuf, out_hbm.at[wid])

    return sc_kernel(scores, k_arr)


def _sm_body(x_ref, t_ref, o_ref):
    x = x_ref[...]
    t = t_ref[...]
    mask = x >= t
    m = jnp.max(x, axis=1, keepdims=True)
    e = jnp.where(mask, jnp.exp(x - m), 0.0)
    o_ref[...] = e / jnp.sum(e, axis=1, keepdims=True)


def _tc_softmax(scores, thresh):
    rows, n = scores.shape
    r_blk = 32
    return pl.pallas_call(
        _sm_body,
        grid=(rows // r_blk,),
        in_specs=[
            pl.BlockSpec((r_blk, n), lambda i: (i, 0)),
            pl.BlockSpec((r_blk, 1), lambda i: (i, 0)),
        ],
        out_specs=pl.BlockSpec((r_blk, n), lambda i: (i, 0)),
        out_shape=jax.ShapeDtypeStruct(scores.shape, scores.dtype),
    )(scores, thresh)


def kernel(scores, k):
    k_arr = jnp.full((16,), k, jnp.int32)
    th = _sc_thresholds(scores, k_arr)
    thresh = th[:, :_RPW].reshape(_ROWS, 1)
    return _tc_softmax(scores, thresh)
